# baseline (speedup 1.0000x reference)
import functools

import jax
import jax.numpy as jnp
from jax import lax
from jax.experimental import pallas as pl
from jax.experimental.pallas import tpu as pltpu
from jax.experimental.pallas import tpu_sc as plsc

_B = 1024
_KEY_D = 768
_EMB_D = 768
_POOL = 100
_E_LEN = 20
_TOP_K = 2
_HALF = _E_LEN // 2
_NROWS = _B * _TOP_K
_RW = _E_LEN * _EMB_D // 128

_NC = 2
_NS = 16
_NW = _NC * _NS
_FPW = _NROWS // _NW
_CHUNK = 4
_NCHUNK = _FPW // _CHUNK

_GBLK = 64


def _select_body(xq_ref, ek_ref, idx_ref, cnt_ref):
    xq = xq_ref[...]
    ek = ek_ref[...]
    nk = ek / jnp.clip(jnp.sqrt(jnp.sum(ek * ek, axis=1, keepdims=True)), 1e-12)
    q = xq / jnp.clip(jnp.sqrt(jnp.sum(xq * xq, axis=1, keepdims=True)), 1e-12)
    cos = lax.dot_general(q, nk, (((1,), (1,)), ((), ())),
                          preferred_element_type=jnp.float32)
    iota = lax.broadcasted_iota(jnp.int32, (_B, _POOL), 1)
    m1 = jnp.max(cos, axis=1, keepdims=True)
    i1 = jnp.min(jnp.where(cos == m1, iota, _POOL), axis=1, keepdims=True)
    cos2 = jnp.where(iota == i1, -jnp.inf, cos)
    m2 = jnp.max(cos2, axis=1, keepdims=True)
    i2 = jnp.min(jnp.where(cos2 == m2, iota, _POOL), axis=1, keepdims=True)
    idx_ref[...] = jnp.concatenate([i1, i2], axis=1)
    cnt = ((iota == i1).astype(jnp.int32) + (iota == i2).astype(jnp.int32))
    cnt_ref[...] = jnp.sum(cnt, axis=0, keepdims=True)


def _select(x_querry, e_k):
    return pl.pallas_call(
        _select_body,
        out_shape=(
            jax.ShapeDtypeStruct((_B, _TOP_K), jnp.int32),
            jax.ShapeDtypeStruct((1, _POOL), jnp.int32),
        ),
    )(x_querry, e_k)


def _gather_body(ep_hbm, idxp_hbm, m_hbm, ia, ib, ra, rb, gsem):
    wid = lax.axis_index("s") * _NC + lax.axis_index("c")
    c0 = wid * _NCHUNK

    def fetch_and_gather(c, idx_v, rows_v):
        pltpu.sync_copy(idxp_hbm.at[c0 + c], idx_v)
        pltpu.async_copy(ep_hbm.at[idx_v], rows_v, gsem)

    def drain(idx_v, rows_v):
        pltpu.make_async_copy(ep_hbm.at[idx_v], rows_v, gsem).wait()

    def write_out(c, rows_v):
        pltpu.sync_copy(rows_v, m_hbm.at[pl.ds((c0 + c) * _CHUNK, _CHUNK)])

    fetch_and_gather(0, ia, ra)

    def step(o, carry):
        ca = 2 * o
        cb = 2 * o + 1
        fetch_and_gather(cb, ib, rb)
        drain(ia, ra)
        write_out(ca, ra)

        @pl.when(o < _NCHUNK // 2 - 1)
        def _():
            fetch_and_gather(ca + 2, ia, ra)

        drain(ib, rb)
        write_out(cb, rb)
        return carry

    lax.fori_loop(0, _NCHUNK // 2, step, 0)


@functools.partial(
    pl.kernel,
    mesh=plsc.VectorSubcoreMesh(core_axis_name="c", subcore_axis_name="s"),
    out_type=jax.ShapeDtypeStruct((_NROWS, _RW, 128), jnp.float32),
    scratch_types=[
        pltpu.VMEM((_CHUNK,), jnp.int32),
        pltpu.VMEM((_CHUNK,), jnp.int32),
        pltpu.VMEM((_CHUNK, _RW, 128), jnp.float32),
        pltpu.VMEM((_CHUNK, _RW, 128), jnp.float32),
        pltpu.SemaphoreType.DMA,
    ],
    compiler_params=pltpu.CompilerParams(use_tc_tiling_on_sc=True),
)
def _gather(ep_hbm, idxp_hbm, m_hbm, ia, ib, ra, rb, gsem):
    _gather_body(ep_hbm, idxp_hbm, m_hbm, ia, ib, ra, rb, gsem)


def _relayout_body(m_ref, ek_ref, ev_ref):
    blk = m_ref[...]
    ek_ref[...] = blk[:, :_RW // 2, :].reshape(_GBLK, _E_LEN, _EMB_D)
    ev_ref[...] = blk[:, _RW // 2:, :].reshape(_GBLK, _E_LEN, _EMB_D)


def _relayout(m):
    n = _B // _GBLK
    spec_in = pl.BlockSpec((_TOP_K * _GBLK, _RW, 128), lambda i: (i, 0, 0))
    spec_out = pl.BlockSpec((_GBLK, _E_LEN, _EMB_D), lambda i: (i, 0, 0))
    return pl.pallas_call(
        _relayout_body,
        grid=(n,),
        in_specs=[spec_in],
        out_specs=(spec_out, spec_out),
        out_shape=(
            jax.ShapeDtypeStruct((_B, _E_LEN, _EMB_D), jnp.float32),
            jax.ShapeDtypeStruct((_B, _E_LEN, _EMB_D), jnp.float32),
        ),
    )(m)


def kernel(x_querry, l, x_block, e_k, e_p):
    k_idx, cnt = _select(x_querry, e_k)
    idxp = k_idx.reshape(_NROWS // _CHUNK, _CHUNK)
    ep_r = e_p.reshape(_POOL, _RW, 128)
    m = _gather(ep_r, idxp)
    one = lax.optimization_barrier((jnp.float32(1.0), k_idx))[0]
    xb = x_block * one
    m2, xb2 = lax.optimization_barrier((m, xb))
    Ek = m2[:, :_RW // 2, :].reshape(_B, _E_LEN, _EMB_D)
    Ev = m2[:, _RW // 2:, :].reshape(_B, _E_LEN, _EMB_D)
    eval_count = cnt.reshape(_POOL)
    return (Ek, Ev, xb2, eval_count)


# speedup vs baseline: 1.1920x; 1.1920x over previous
import functools

import jax
import jax.numpy as jnp
from jax import lax
from jax.experimental import pallas as pl
from jax.experimental.pallas import tpu as pltpu
from jax.experimental.pallas import tpu_sc as plsc

_B = 1024
_KEY_D = 768
_EMB_D = 768
_POOL = 100
_E_LEN = 20
_TOP_K = 2
_HALF = _E_LEN // 2
_NROWS = _B * _TOP_K
_RW = _E_LEN * _EMB_D // 128

_NC = 2
_NS = 16
_NW = _NC * _NS
_FPW = _NROWS // _NW
_CHUNK = 4
_NCHUNK = _FPW // _CHUNK

_GBLK = 64


def _select_body(xq_ref, ek_ref, idx_ref, cnt_ref):
    xq = xq_ref[...]
    ek = ek_ref[...]
    nk = ek / jnp.clip(jnp.sqrt(jnp.sum(ek * ek, axis=1, keepdims=True)), 1e-12)
    q = xq / jnp.clip(jnp.sqrt(jnp.sum(xq * xq, axis=1, keepdims=True)), 1e-12)
    cos = lax.dot_general(q, nk, (((1,), (1,)), ((), ())),
                          preferred_element_type=jnp.float32)
    iota = lax.broadcasted_iota(jnp.int32, (_B, _POOL), 1)
    m1 = jnp.max(cos, axis=1, keepdims=True)
    i1 = jnp.min(jnp.where(cos == m1, iota, _POOL), axis=1, keepdims=True)
    cos2 = jnp.where(iota == i1, -jnp.inf, cos)
    m2 = jnp.max(cos2, axis=1, keepdims=True)
    i2 = jnp.min(jnp.where(cos2 == m2, iota, _POOL), axis=1, keepdims=True)
    idx_ref[...] = jnp.concatenate([i1, i2], axis=1)
    cnt = ((iota == i1).astype(jnp.int32) + (iota == i2).astype(jnp.int32))
    cnt_ref[...] = jnp.sum(cnt, axis=0, keepdims=True)


def _select(x_querry, e_k):
    return pl.pallas_call(
        _select_body,
        out_shape=(
            jax.ShapeDtypeStruct((_B, _TOP_K), jnp.int32),
            jax.ShapeDtypeStruct((1, _POOL), jnp.int32),
        ),
    )(x_querry, e_k)


def _gather_body(ep_hbm, idxp_hbm, m_hbm, ia, ib, ra, rb, gsem):
    wid = lax.axis_index("s") * _NC + lax.axis_index("c")
    c0 = wid * _NCHUNK

    def fetch_and_gather(c, idx_v, rows_v):
        pltpu.sync_copy(idxp_hbm.at[c0 + c], idx_v)
        pltpu.async_copy(ep_hbm.at[idx_v], rows_v, gsem)

    def drain(idx_v, rows_v):
        pltpu.make_async_copy(ep_hbm.at[idx_v], rows_v, gsem).wait()

    def write_out(c, rows_v):
        pltpu.sync_copy(rows_v, m_hbm.at[pl.ds((c0 + c) * _CHUNK, _CHUNK)])

    fetch_and_gather(0, ia, ra)

    def step(o, carry):
        ca = 2 * o
        cb = 2 * o + 1
        fetch_and_gather(cb, ib, rb)
        drain(ia, ra)
        write_out(ca, ra)

        @pl.when(o < _NCHUNK // 2 - 1)
        def _():
            fetch_and_gather(ca + 2, ia, ra)

        drain(ib, rb)
        write_out(cb, rb)
        return carry

    lax.fori_loop(0, _NCHUNK // 2, step, 0)


@functools.partial(
    pl.kernel,
    mesh=plsc.VectorSubcoreMesh(core_axis_name="c", subcore_axis_name="s"),
    out_type=jax.ShapeDtypeStruct((_NROWS, _RW, 128), jnp.float32),
    scratch_types=[
        pltpu.VMEM((_CHUNK,), jnp.int32),
        pltpu.VMEM((_CHUNK,), jnp.int32),
        pltpu.VMEM((_CHUNK, _RW, 128), jnp.float32),
        pltpu.VMEM((_CHUNK, _RW, 128), jnp.float32),
        pltpu.SemaphoreType.DMA,
    ],
    compiler_params=pltpu.CompilerParams(use_tc_tiling_on_sc=True),
)
def _gather(ep_hbm, idxp_hbm, m_hbm, ia, ib, ra, rb, gsem):
    _gather_body(ep_hbm, idxp_hbm, m_hbm, ia, ib, ra, rb, gsem)


def _relayout_body(m_ref, ek_ref, ev_ref):
    blk = m_ref[...]
    ek_ref[...] = blk[:, :_RW // 2, :].reshape(_GBLK, _E_LEN, _EMB_D)
    ev_ref[...] = blk[:, _RW // 2:, :].reshape(_GBLK, _E_LEN, _EMB_D)


def _relayout(m):
    n = _B // _GBLK
    spec_in = pl.BlockSpec((_TOP_K * _GBLK, _RW, 128), lambda i: (i, 0, 0))
    spec_out = pl.BlockSpec((_GBLK, _E_LEN, _EMB_D), lambda i: (i, 0, 0))
    return pl.pallas_call(
        _relayout_body,
        grid=(n,),
        in_specs=[spec_in],
        out_specs=(spec_out, spec_out),
        out_shape=(
            jax.ShapeDtypeStruct((_B, _E_LEN, _EMB_D), jnp.float32),
            jax.ShapeDtypeStruct((_B, _E_LEN, _EMB_D), jnp.float32),
        ),
    )(m)


def kernel(x_querry, l, x_block, e_k, e_p):
    k_idx, cnt = _select(x_querry, e_k)
    idxp = k_idx.reshape(_NROWS // _CHUNK, _CHUNK)
    ep_r = e_p.reshape(_POOL, _RW, 128)
    m = _gather(ep_r, idxp)
    one = lax.optimization_barrier((jnp.float32(1.0), k_idx))[0]
    xb = x_block * one
    m2, xb2 = lax.optimization_barrier((m, xb))
    Ek, Ev = _relayout(m2)
    eval_count = cnt.reshape(_POOL)
    return (Ek, Ev, xb2, eval_count)
